# baseline (speedup 1.0000x reference)
import functools

import jax
import jax.numpy as jnp
from jax import lax
from jax.experimental import pallas as pl
from jax.experimental.pallas import tpu as pltpu
from jax.experimental.pallas import tpu_sc as plsc

NC = 2
NS = 16
NW = NC * NS
L = 16
U = 4

H = 1024
HPAD = H + 16
HB3 = 2048
SPEC_LO = 0.68
SPEC_HI = 0.72


def _srl(x, n):
  return lax.shift_right_logical(x, jnp.full(jnp.shape(x), n, jnp.int32))


def _bits(x):
  return lax.bitcast_convert_type(x, jnp.int32)


def _make_sc_kernel(B, P, K):
  rows_per_w = B // NW
  mesh = plsc.VectorSubcoreMesh(core_axis_name="c", subcore_axis_name="s")

  @functools.partial(
      pl.kernel,
      out_type=jax.ShapeDtypeStruct((NW * L,), jnp.float32),
      mesh=mesh,
      compiler_params=pltpu.CompilerParams(needs_layout_passes=False),
      scratch_types=[
          pltpu.VMEM((P,), jnp.float32),
          pltpu.VMEM((P,), jnp.float32),
          pltpu.VMEM((HPAD,), jnp.int32),
          pltpu.VMEM((HPAD,), jnp.int32),
          pltpu.VMEM((HB3,), jnp.int32),
          pltpu.VMEM((L,), jnp.float32),
      ],
  )
  def sc_kernel(loss_hbm, out_hbm, row_v, buf_v, c1, c2, c3, outv):
    cid = lax.axis_index("c")
    sid = lax.axis_index("s")
    wid = sid * NC + cid
    iota = lax.iota(jnp.int32, L)
    ones_i = jnp.ones((L,), jnp.int32)
    zeros_i = jnp.zeros((L,), jnp.int32)
    zeros_f = jnp.zeros((L,), jnp.float32)
    last_lane = jnp.full((L,), L - 1, jnp.int32)

    def bc_last(v):
      return v.at[last_lane].get(mode="promise_in_bounds")

    def zero_ref(ref, n):
      def zbody(i, _):
        ref[pl.ds(i * L, L)] = zeros_i
        return 0

      lax.fori_loop(0, n // L, zbody, 0)

    zero_ref(c1, HPAD)
    zero_ref(c2, HPAD)
    zero_ref(c3, HB3)

    def find_count(c_ref, nchunks, k_rem):
      def fbody(i, carry):
        cum, found, cum_sel, base_sel, c_sel = carry
        c = c_ref[pl.ds(i * L, L)]
        tot = jnp.sum(c)
        c_ref[pl.ds(i * L, L)] = zeros_i
        hit = jnp.logical_and(jnp.logical_not(found), (cum + tot) >= k_rem)
        cum_sel = jnp.where(hit, cum, cum_sel)
        base_sel = jnp.where(hit, i * L, base_sel)
        c_sel = jnp.where(hit, c, c_sel)
        return (cum + tot, jnp.logical_or(found, hit), cum_sel, base_sel,
                c_sel)

      init = (jnp.int32(0), jnp.bool_(False), jnp.int32(0), jnp.int32(0),
              zeros_i)
      _, _, cum_sel, base_sel, c_sel = lax.fori_loop(0, nchunks, fbody, init)
      scan_c = plsc.cumsum(c_sel)
      cross = (cum_sel + scan_c) >= k_rem
      lane = jnp.min(jnp.where(cross, iota, L - 1))
      nb = cum_sel + jnp.sum(jnp.where(iota < lane, c_sel, 0))
      return base_sel + lane, nb

    def hist_pass(src, m, qfn, c_ref):
      full = isinstance(m, int)
      nch = m // L if full else _srl(m + (L - 1), 4)

      @plsc.parallel_loop(0, nch, unroll=2 * U)
      def _(i):
        base = i * L
        x = src[pl.ds(base, L)]
        b = qfn(x)
        if full:
          plsc.addupdate_scatter(c_ref, [b], ones_i)
        else:
          valid = (base + iota) < m
          plsc.addupdate_scatter(c_ref, [b], ones_i, mask=valid)

    def compact_pass(src, dst, m, classify, c_next, below_slot=None):
      full = isinstance(m, int)
      nit = m // (L * U) if full else _srl(m + (L * U - 1), 6)

      @plsc.parallel_loop(0, nit, unroll=2, carry=(zeros_i, zeros_f))
      def res(i, carry):
        offv, sacc = carry
        xs, sels, pms, cnts, keys, bels = [], [], [], [], [], []
        for u in range(U):
          base = (i * U + u) * L
          x = src[pl.ds(base, L)]
          below, sel, nkey = classify(x)
          if not full:
            valid = (base + iota) < m
            below = jnp.logical_and(valid, below)
            sel = jnp.logical_and(valid, sel)
          sacc = sacc + jnp.where(below, x, jnp.float32(0))
          pm = plsc.cumsum(sel.astype(jnp.int32))
          xs.append(x)
          sels.append(sel)
          pms.append(pm)
          cnts.append(bc_last(pm))
          keys.append(nkey)
          bels.append(below)
        c01 = cnts[0] + cnts[1]
        offs = [offv, offv + cnts[0], offv + c01, offv + c01 + cnts[2]]
        for u in range(U):
          plsc.store_scatter(dst, [offs[u] + pms[u] - 1], xs[u], mask=sels[u])
          if c_next is not None:
            plsc.addupdate_scatter(c_next, [keys[u]], ones_i, mask=sels[u])
          if below_slot is not None:
            plsc.addupdate_scatter(c_next, [below_slot], ones_i, mask=bels[u])
        return (offs[3] + cnts[3], sacc)

      offv, sacc = res
      return jnp.max(offv), sacc

    def below_sum(src, m, t_bits):
      @plsc.parallel_loop(0, _srl(m + (L - 1), 4), unroll=U, carry=zeros_f)
      def res(i, sacc):
        base = i * L
        x = src[pl.ds(base, L)]
        below = jnp.logical_and((base + iota) < m, _bits(x) < t_bits)
        return sacc + jnp.where(below, x, jnp.float32(0))

      return res

    r2 = lambda x: (x * jnp.float32(H * H)).astype(jnp.int32)
    qb1 = lambda x: _srl(_bits(x), 21) & 0x3FF
    qb2 = lambda x: _srl(_bits(x), 11) & 0x3FF
    qb3 = lambda x: _bits(x) & 0x7FF

    def classify_q(qfn, bs, nqfn):
      def f(x):
        q = qfn(x)
        return q < bs, q == bs, None if nqfn is None else nqfn(x)

      return f

    def fast_tail(m3, k_rem2, sacc_lin):
      x = row_v[pl.ds(0, L)]
      xk = jnp.where(iota < m3, x, jnp.float32(2.0))
      xs = lax.sort(xk)
      tail = jnp.sum(jnp.where(iota < k_rem2, xs, jnp.float32(0)))
      return jnp.sum(sacc_lin) + tail

    def slow_tail(m3, k_rem2, sacc_lin):
      hist_pass(row_v, m3, qb1, c1)
      b1, nb1 = find_count(c1, HPAD // L, k_rem2)
      m4, sacc3 = compact_pass(row_v, buf_v, m3, classify_q(qb1, b1, qb2), c2)
      b2, nb2 = find_count(c2, HPAD // L, k_rem2 - nb1)
      m5, sacc4 = compact_pass(buf_v, row_v, m4, classify_q(qb2, b2, qb3), c3)
      b3, nb3 = find_count(c3, HB3 // L, k_rem2 - nb1 - nb2)
      t_bits = (b1 << 21) | (b2 << 11) | b3
      sacc5 = below_sum(row_v, m5, t_bits)
      t_vec = lax.bitcast_convert_type(jnp.full((L,), t_bits, jnp.int32),
                                       jnp.float32)
      rem = (k_rem2 - nb1 - nb2 - nb3).astype(jnp.float32)
      contrib_v = (sacc_lin + sacc3 + sacc4 + sacc5
                   + rem * t_vec * jnp.float32(1.0 / L))
      return jnp.sum(contrib_v)

    spec_scale = jnp.float32(H / (SPEC_HI - SPEC_LO))
    spec_shift = jnp.float32(-SPEC_LO * H / (SPEC_HI - SPEC_LO))
    below_slot_v = jnp.full((L,), H + 8, jnp.int32)

    def q_spec(x):
      return (x * spec_scale + spec_shift).astype(jnp.int32)

    def cls_spec(x):
      q = q_spec(x)
      sel = lax.bitcast_convert_type(q, jnp.uint32) < jnp.uint32(H)
      return q < 0, sel, q

    def committed(m_int, k_rem, sacc_a):
      b2, nb2 = find_count(c2, H // L, k_rem)

      def cls2s(x):
        q = q_spec(x)
        return q < b2, q == b2, None

      m3, sacc2 = compact_pass(buf_v, row_v, m_int, cls2s, None)
      return lax.cond(m3 <= L, fast_tail, slow_tail, m3, k_rem - nb2,
                      sacc_a + sacc2)

    def fallback(m_int, k_rem, sacc_a):
      del m_int, k_rem, sacc_a
      zero_ref(c2, HPAD)

      hist_pass(row_v, P, lambda x: _srl(r2(x), 10), c1)
      b1, nb1 = find_count(c1, HPAD // L, jnp.int32(K))
      base1 = b1 * H

      def cls1(x):
        d = r2(x) - base1
        sel = lax.bitcast_convert_type(d, jnp.uint32) < jnp.uint32(H)
        return d < 0, sel, d

      m2, sacc1 = compact_pass(row_v, buf_v, P, cls1, c2)

      b2, nb2 = find_count(c2, HPAD // L, K - nb1)
      base2 = base1 + b2

      def cls2(x):
        d = r2(x) - base2
        return d < 0, d == 0, None

      m3, sacc2 = compact_pass(buf_v, row_v, m2, cls2, None)

      k_rem2 = K - nb1 - nb2
      return lax.cond(m3 <= L, fast_tail, slow_tail, m3, k_rem2,
                      sacc1 + sacc2)

    def row_body(r, contrib_acc):
      row = wid * rows_per_w + r
      pltpu.sync_copy(loss_hbm.at[row], row_v)

      m_int, sacc_a = compact_pass(row_v, buf_v, P, cls_spec, c2,
                                   below_slot=below_slot_v)
      nb_chunk = c2[pl.ds(H, L)]
      n_below = jnp.sum(jnp.where(iota == 8, nb_chunk, 0))
      c2[pl.ds(H, L)] = zeros_i
      k_rem = K - n_below
      commit = jnp.logical_and(k_rem >= 1, k_rem <= m_int)

      contrib = lax.cond(commit, committed, fallback, m_int, k_rem, sacc_a)
      return jnp.where(iota == r, contrib, contrib_acc)

    contrib_acc = lax.fori_loop(0, rows_per_w, row_body, zeros_f)
    outv[...] = contrib_acc
    pltpu.sync_copy(outv, out_hbm.at[pl.ds(wid * L, L)])

  return sc_kernel


def _tc_mean(x_ref, o_ref, *, scale):
  o_ref[...] = jnp.sum(x_ref[...], keepdims=True).reshape(1, 1) * scale


def kernel(loss):
  B = loss.shape[0]
  P = loss.reshape(B, -1).shape[1]
  K = int(0.7 * P)
  sc_kernel = _make_sc_kernel(B, P, K)
  partials = sc_kernel(loss.reshape(B, -1))
  out = pl.pallas_call(
      functools.partial(_tc_mean, scale=1.0 / (B * K)),
      out_shape=jax.ShapeDtypeStruct((1, 1), jnp.float32),
  )(partials.reshape(4, NW * L // 4))
  return out[0, 0]


# speedup vs baseline: 1.3575x; 1.3575x over previous
import functools

import jax
import jax.numpy as jnp
from jax import lax
from jax.experimental import pallas as pl
from jax.experimental.pallas import tpu as pltpu
from jax.experimental.pallas import tpu_sc as plsc

NC = 2
NS = 16
NW = NC * NS
L = 16
U = 4

H = 1024
HPAD = H + 16
HB3 = 2048
SPEC_LO = 0.68
SPEC_HI = 0.72


def _srl(x, n):
  return lax.shift_right_logical(x, jnp.full(jnp.shape(x), n, jnp.int32))


def _bits(x):
  return lax.bitcast_convert_type(x, jnp.int32)


def _make_sc_kernel(B, P, K):
  rows_per_w = B // NW
  mesh = plsc.VectorSubcoreMesh(core_axis_name="c", subcore_axis_name="s")

  @functools.partial(
      pl.kernel,
      out_type=jax.ShapeDtypeStruct((NW * L,), jnp.float32),
      mesh=mesh,
      compiler_params=pltpu.CompilerParams(needs_layout_passes=False),
      scratch_types=[
          pltpu.VMEM((P,), jnp.float32),
          pltpu.VMEM((P,), jnp.float32),
          pltpu.VMEM((HPAD,), jnp.int32),
          pltpu.VMEM((HPAD,), jnp.int32),
          pltpu.VMEM((HB3,), jnp.int32),
          pltpu.VMEM((L,), jnp.float32),
      ],
  )
  def sc_kernel(loss_hbm, out_hbm, row_v, buf_v, c1, c2, c3, outv):
    cid = lax.axis_index("c")
    sid = lax.axis_index("s")
    wid = sid * NC + cid
    iota = lax.iota(jnp.int32, L)
    ones_i = jnp.ones((L,), jnp.int32)
    zeros_i = jnp.zeros((L,), jnp.int32)
    zeros_f = jnp.zeros((L,), jnp.float32)
    last_lane = jnp.full((L,), L - 1, jnp.int32)

    def bc_last(v):
      return v.at[last_lane].get(mode="promise_in_bounds")

    def zero_ref(ref, n):
      def zbody(i, _):
        ref[pl.ds(i * L, L)] = zeros_i
        return 0

      lax.fori_loop(0, n // L, zbody, 0)

    zero_ref(c1, HPAD)
    zero_ref(c2, HPAD)
    zero_ref(c3, HB3)

    def find_count(c_ref, nchunks, k_rem):
      def fbody(i, carry):
        cum, found, cum_sel, base_sel, c_sel = carry
        c = c_ref[pl.ds(i * L, L)]
        tot = jnp.sum(c)
        c_ref[pl.ds(i * L, L)] = zeros_i
        hit = jnp.logical_and(jnp.logical_not(found), (cum + tot) >= k_rem)
        cum_sel = jnp.where(hit, cum, cum_sel)
        base_sel = jnp.where(hit, i * L, base_sel)
        c_sel = jnp.where(hit, c, c_sel)
        return (cum + tot, jnp.logical_or(found, hit), cum_sel, base_sel,
                c_sel)

      init = (jnp.int32(0), jnp.bool_(False), jnp.int32(0), jnp.int32(0),
              zeros_i)
      _, _, cum_sel, base_sel, c_sel = lax.fori_loop(0, nchunks, fbody, init)
      scan_c = plsc.cumsum(c_sel)
      cross = (cum_sel + scan_c) >= k_rem
      lane = jnp.min(jnp.where(cross, iota, L - 1))
      nb = cum_sel + jnp.sum(jnp.where(iota < lane, c_sel, 0))
      return base_sel + lane, nb

    def hist_pass(src, m, qfn, c_ref):
      full = isinstance(m, int)
      nch = m // L if full else _srl(m + (L - 1), 4)

      @plsc.parallel_loop(0, nch, unroll=2 * U)
      def _(i):
        base = i * L
        x = src[pl.ds(base, L)]
        b = qfn(x)
        if full:
          plsc.addupdate_scatter(c_ref, [b], ones_i)
        else:
          valid = (base + iota) < m
          plsc.addupdate_scatter(c_ref, [b], ones_i, mask=valid)

    def compact_pass(src, dst, m, classify, c_next, count_below=False):
      full = isinstance(m, int)
      nit = m // (L * U) if full else _srl(m + (L * U - 1), 6)
      init = (zeros_i, zeros_f, zeros_i) if count_below else (zeros_i, zeros_f)

      @plsc.parallel_loop(0, nit, unroll=2, carry=init)
      def res(i, carry):
        offv, sacc = carry[0], carry[1]
        cb = carry[2] if count_below else None
        xs, sels, pms, cnts, keys = [], [], [], [], []
        for u in range(U):
          base = (i * U + u) * L
          x = src[pl.ds(base, L)]
          below, sel, nkey = classify(x)
          if not full:
            valid = (base + iota) < m
            below = jnp.logical_and(valid, below)
            sel = jnp.logical_and(valid, sel)
          sacc = sacc + jnp.where(below, x, jnp.float32(0))
          if count_below:
            cb = cb + below.astype(jnp.int32)
          pm = plsc.cumsum(sel.astype(jnp.int32))
          xs.append(x)
          sels.append(sel)
          pms.append(pm)
          cnts.append(bc_last(pm))
          keys.append(nkey)
        c01 = cnts[0] + cnts[1]
        offs = [offv, offv + cnts[0], offv + c01, offv + c01 + cnts[2]]
        for u in range(U):
          plsc.store_scatter(dst, [offs[u] + pms[u] - 1], xs[u], mask=sels[u])
          if c_next is not None:
            plsc.addupdate_scatter(c_next, [keys[u]], ones_i, mask=sels[u])
        if count_below:
          return (offs[3] + cnts[3], sacc, cb)
        return (offs[3] + cnts[3], sacc)

      if count_below:
        offv, sacc, cb = res
        return jnp.max(offv), sacc, jnp.sum(cb)
      offv, sacc = res
      return jnp.max(offv), sacc

    def below_sum(src, m, t_bits):
      @plsc.parallel_loop(0, _srl(m + (L - 1), 4), unroll=U, carry=zeros_f)
      def res(i, sacc):
        base = i * L
        x = src[pl.ds(base, L)]
        below = jnp.logical_and((base + iota) < m, _bits(x) < t_bits)
        return sacc + jnp.where(below, x, jnp.float32(0))

      return res

    r2 = lambda x: (x * jnp.float32(H * H)).astype(jnp.int32)
    qb1 = lambda x: _srl(_bits(x), 21) & 0x3FF
    qb2 = lambda x: _srl(_bits(x), 11) & 0x3FF
    qb3 = lambda x: _bits(x) & 0x7FF

    def classify_q(qfn, bs, nqfn):
      def f(x):
        q = qfn(x)
        return q < bs, q == bs, None if nqfn is None else nqfn(x)

      return f

    def fast_tail(m3, k_rem2, sacc_lin):
      x = row_v[pl.ds(0, L)]
      xk = jnp.where(iota < m3, x, jnp.float32(2.0))
      xs = lax.sort(xk)
      tail = jnp.sum(jnp.where(iota < k_rem2, xs, jnp.float32(0)))
      return jnp.sum(sacc_lin) + tail

    def slow_tail(m3, k_rem2, sacc_lin):
      hist_pass(row_v, m3, qb1, c1)
      b1, nb1 = find_count(c1, HPAD // L, k_rem2)
      m4, sacc3 = compact_pass(row_v, buf_v, m3, classify_q(qb1, b1, qb2), c2)
      b2, nb2 = find_count(c2, HPAD // L, k_rem2 - nb1)
      m5, sacc4 = compact_pass(buf_v, row_v, m4, classify_q(qb2, b2, qb3), c3)
      b3, nb3 = find_count(c3, HB3 // L, k_rem2 - nb1 - nb2)
      t_bits = (b1 << 21) | (b2 << 11) | b3
      sacc5 = below_sum(row_v, m5, t_bits)
      t_vec = lax.bitcast_convert_type(jnp.full((L,), t_bits, jnp.int32),
                                       jnp.float32)
      rem = (k_rem2 - nb1 - nb2 - nb3).astype(jnp.float32)
      contrib_v = (sacc_lin + sacc3 + sacc4 + sacc5
                   + rem * t_vec * jnp.float32(1.0 / L))
      return jnp.sum(contrib_v)

    spec_scale = jnp.float32(H / (SPEC_HI - SPEC_LO))
    spec_shift = jnp.float32(-SPEC_LO * H / (SPEC_HI - SPEC_LO))

    def q_spec(x):
      return (x * spec_scale + spec_shift).astype(jnp.int32)

    def cls_spec(x):
      q = q_spec(x)
      sel = lax.bitcast_convert_type(q, jnp.uint32) < jnp.uint32(H)
      return q < 0, sel, q

    def row_body(r, contrib_acc):
      row = wid * rows_per_w + r
      pltpu.sync_copy(loss_hbm.at[row], row_v)

      m_int, sacc_a, n_below = compact_pass(row_v, buf_v, P, cls_spec, c2,
                                            count_below=True)
      k_rem = K - n_below
      commit = jnp.logical_and(k_rem >= 1, k_rem <= m_int)

      b2s, nb2s = find_count(c2, H // L, k_rem)

      def cls2s(x):
        q = q_spec(x)
        return q < b2s, q == b2s, None

      m3s, sacc2s = compact_pass(buf_v, row_v, jnp.where(commit, m_int, 0),
                                 cls2s, None)
      contrib_s = lax.cond(m3s <= L, fast_tail, slow_tail, m3s, k_rem - nb2s,
                           sacc_a + sacc2s)

      m_fb = jnp.where(commit, jnp.int32(0), jnp.int32(P))
      hist_pass(row_v, m_fb, lambda x: _srl(r2(x), 10), c1)
      b1, nb1 = find_count(c1, HPAD // L, jnp.int32(K))
      base1 = b1 * H

      def cls1(x):
        d = r2(x) - base1
        sel = lax.bitcast_convert_type(d, jnp.uint32) < jnp.uint32(H)
        return d < 0, sel, d

      m2, sacc1 = compact_pass(row_v, buf_v, m_fb, cls1, c2)

      b2, nb2 = find_count(c2, HPAD // L, K - nb1)
      base2 = base1 + b2

      def cls2(x):
        d = r2(x) - base2
        return d < 0, d == 0, None

      m3, sacc2 = compact_pass(buf_v, row_v, m2, cls2, None)

      k_rem2 = K - nb1 - nb2
      contrib_f = lax.cond(m3 <= L, fast_tail, slow_tail, m3, k_rem2,
                           sacc1 + sacc2)

      contrib = jnp.where(commit, contrib_s, contrib_f)
      return jnp.where(iota == r, contrib, contrib_acc)

    contrib_acc = lax.fori_loop(0, rows_per_w, row_body, zeros_f)
    outv[...] = contrib_acc
    pltpu.sync_copy(outv, out_hbm.at[pl.ds(wid * L, L)])

  return sc_kernel


def _tc_mean(x_ref, o_ref, *, scale):
  o_ref[...] = jnp.sum(x_ref[...], keepdims=True).reshape(1, 1) * scale


def kernel(loss):
  B = loss.shape[0]
  P = loss.reshape(B, -1).shape[1]
  K = int(0.7 * P)
  sc_kernel = _make_sc_kernel(B, P, K)
  partials = sc_kernel(loss.reshape(B, -1))
  out = pl.pallas_call(
      functools.partial(_tc_mean, scale=1.0 / (B * K)),
      out_shape=jax.ShapeDtypeStruct((1, 1), jnp.float32),
  )(partials.reshape(4, NW * L // 4))
  return out[0, 0]


# speedup vs baseline: 1.4134x; 1.0412x over previous
import functools

import jax
import jax.numpy as jnp
from jax import lax
from jax.experimental import pallas as pl
from jax.experimental.pallas import tpu as pltpu
from jax.experimental.pallas import tpu_sc as plsc

NC = 2
NS = 16
NW = NC * NS
L = 16
U = 4

H = 1024
HPAD = H + 16
HB3 = 2048
SPEC_LO = 0.68
SPEC_HI = 0.72


def _srl(x, n):
  return lax.shift_right_logical(x, jnp.full(jnp.shape(x), n, jnp.int32))


def _bits(x):
  return lax.bitcast_convert_type(x, jnp.int32)


def _make_sc_kernel(B, P, K):
  rows_per_w = B // NW
  mesh = plsc.VectorSubcoreMesh(core_axis_name="c", subcore_axis_name="s")

  @functools.partial(
      pl.kernel,
      out_type=jax.ShapeDtypeStruct((NW * L,), jnp.float32),
      mesh=mesh,
      compiler_params=pltpu.CompilerParams(needs_layout_passes=False),
      scratch_types=[
          pltpu.VMEM((P,), jnp.float32),
          pltpu.VMEM((P,), jnp.float32),
          pltpu.VMEM((HPAD,), jnp.int32),
          pltpu.VMEM((HPAD,), jnp.int32),
          pltpu.VMEM((HB3,), jnp.int32),
          pltpu.VMEM((L,), jnp.float32),
      ],
  )
  def sc_kernel(loss_hbm, out_hbm, row_v, buf_v, c1, c2, c3, outv):
    cid = lax.axis_index("c")
    sid = lax.axis_index("s")
    wid = sid * NC + cid
    iota = lax.iota(jnp.int32, L)
    ones_i = jnp.ones((L,), jnp.int32)
    zeros_i = jnp.zeros((L,), jnp.int32)
    zeros_f = jnp.zeros((L,), jnp.float32)
    last_lane = jnp.full((L,), L - 1, jnp.int32)

    def bc_last(v):
      return v.at[last_lane].get(mode="promise_in_bounds")

    def zero_ref(ref, n):
      def zbody(i, _):
        ref[pl.ds(i * L, L)] = zeros_i
        return 0

      lax.fori_loop(0, n // L, zbody, 0)

    zero_ref(c1, HPAD)
    zero_ref(c2, HPAD)
    zero_ref(c3, HB3)

    def find_count(c_ref, nchunks, k_rem):
      def fbody(i, carry):
        cum, found, cum_sel, base_sel, c_sel = carry
        c = c_ref[pl.ds(i * L, L)]
        tot = jnp.sum(c)
        c_ref[pl.ds(i * L, L)] = zeros_i
        hit = jnp.logical_and(jnp.logical_not(found), (cum + tot) >= k_rem)
        cum_sel = jnp.where(hit, cum, cum_sel)
        base_sel = jnp.where(hit, i * L, base_sel)
        c_sel = jnp.where(hit, c, c_sel)
        return (cum + tot, jnp.logical_or(found, hit), cum_sel, base_sel,
                c_sel)

      init = (jnp.int32(0), jnp.bool_(False), jnp.int32(0), jnp.int32(0),
              zeros_i)
      _, _, cum_sel, base_sel, c_sel = lax.fori_loop(0, nchunks, fbody, init)
      scan_c = plsc.cumsum(c_sel)
      cross = (cum_sel + scan_c) >= k_rem
      lane = jnp.min(jnp.where(cross, iota, L - 1))
      nb = cum_sel + jnp.sum(jnp.where(iota < lane, c_sel, 0))
      return base_sel + lane, nb

    def hist_pass(src, m, qfn, c_ref):
      full = isinstance(m, int)
      nch = m // L if full else _srl(m + (L - 1), 4)

      @plsc.parallel_loop(0, nch, unroll=2 * U)
      def _(i):
        base = i * L
        x = src[pl.ds(base, L)]
        b = qfn(x)
        if full:
          plsc.addupdate_scatter(c_ref, [b], ones_i)
        else:
          valid = (base + iota) < m
          plsc.addupdate_scatter(c_ref, [b], ones_i, mask=valid)

    def compact_pass(src, dst, m, classify, c_next, count_below=False):
      full = isinstance(m, int)
      nit = m // (L * U) if full else _srl(m + (L * U - 1), 6)
      init = (zeros_i, zeros_f, zeros_i) if count_below else (zeros_i, zeros_f)

      @plsc.parallel_loop(0, nit, unroll=2, carry=init)
      def res(i, carry):
        offv, sacc = carry[0], carry[1]
        cb = carry[2] if count_below else None
        xs, sels, pms, cnts, keys = [], [], [], [], []
        for u in range(U):
          base = (i * U + u) * L
          x = src[pl.ds(base, L)]
          below, sel, nkey = classify(x)
          if not full:
            valid = (base + iota) < m
            below = jnp.logical_and(valid, below)
            sel = jnp.logical_and(valid, sel)
          sacc = sacc + jnp.where(below, x, jnp.float32(0))
          if count_below:
            cb = cb + below.astype(jnp.int32)
          pm = plsc.cumsum(sel.astype(jnp.int32))
          xs.append(x)
          sels.append(sel)
          pms.append(pm)
          cnts.append(bc_last(pm))
          keys.append(nkey)
        c01 = cnts[0] + cnts[1]
        offs = [offv, offv + cnts[0], offv + c01, offv + c01 + cnts[2]]
        for u in range(U):
          plsc.store_scatter(dst, [offs[u] + pms[u] - 1], xs[u], mask=sels[u])
          if c_next is not None:
            plsc.addupdate_scatter(c_next, [keys[u]], ones_i, mask=sels[u])
        if count_below:
          return (offs[3] + cnts[3], sacc, cb)
        return (offs[3] + cnts[3], sacc)

      if count_below:
        offv, sacc, cb = res
        return jnp.max(offv), sacc, jnp.sum(cb)
      offv, sacc = res
      return jnp.max(offv), sacc

    def below_sum(src, m, t_bits):
      @plsc.parallel_loop(0, _srl(m + (L - 1), 4), unroll=U, carry=zeros_f)
      def res(i, sacc):
        base = i * L
        x = src[pl.ds(base, L)]
        below = jnp.logical_and((base + iota) < m, _bits(x) < t_bits)
        return sacc + jnp.where(below, x, jnp.float32(0))

      return res

    r2 = lambda x: (x * jnp.float32(H * H)).astype(jnp.int32)
    qb1 = lambda x: _srl(_bits(x), 21) & 0x3FF
    qb2 = lambda x: _srl(_bits(x), 11) & 0x3FF
    qb3 = lambda x: _bits(x) & 0x7FF

    def classify_q(qfn, bs, nqfn):
      def f(x):
        q = qfn(x)
        return q < bs, q == bs, None if nqfn is None else nqfn(x)

      return f

    def fast_tail(m3, k_rem2, sacc_lin):
      x = row_v[pl.ds(0, L)]
      xk = jnp.where(iota < m3, x, jnp.float32(2.0))
      xs = lax.sort(xk)
      tail = jnp.sum(jnp.where(iota < k_rem2, xs, jnp.float32(0)))
      return jnp.sum(sacc_lin) + tail

    def slow_tail(m3, k_rem2, sacc_lin):
      live = m3 > L
      m3 = jnp.where(live, m3, 0)
      nh = jnp.where(live, HPAD // L, 0)
      nh3 = jnp.where(live, HB3 // L, 0)
      hist_pass(row_v, m3, qb1, c1)
      b1, nb1 = find_count(c1, nh, k_rem2)
      m4, sacc3 = compact_pass(row_v, buf_v, m3, classify_q(qb1, b1, qb2), c2)
      b2, nb2 = find_count(c2, nh, k_rem2 - nb1)
      m5, sacc4 = compact_pass(buf_v, row_v, m4, classify_q(qb2, b2, qb3), c3)
      b3, nb3 = find_count(c3, nh3, k_rem2 - nb1 - nb2)
      t_bits = (b1 << 21) | (b2 << 11) | b3
      sacc5 = below_sum(row_v, m5, t_bits)
      t_vec = lax.bitcast_convert_type(jnp.full((L,), t_bits, jnp.int32),
                                       jnp.float32)
      rem = (k_rem2 - nb1 - nb2 - nb3).astype(jnp.float32)
      contrib_v = (sacc_lin + sacc3 + sacc4 + sacc5
                   + rem * t_vec * jnp.float32(1.0 / L))
      return jnp.sum(contrib_v)

    spec_scale = jnp.float32(H / (SPEC_HI - SPEC_LO))
    spec_shift = jnp.float32(-SPEC_LO * H / (SPEC_HI - SPEC_LO))

    def q_spec(x):
      return (x * spec_scale + spec_shift).astype(jnp.int32)

    def cls_spec(x):
      q = q_spec(x)
      sel = lax.bitcast_convert_type(q, jnp.uint32) < jnp.uint32(H)
      return q < 0, sel, q

    def row_body(r, contrib_acc):
      row = wid * rows_per_w + r
      pltpu.sync_copy(loss_hbm.at[row], row_v)

      m_int, sacc_a, n_below = compact_pass(row_v, buf_v, P, cls_spec, c2,
                                            count_below=True)
      k_rem = K - n_below
      commit = jnp.logical_and(k_rem >= 1, k_rem <= m_int)

      b2s, nb2s = find_count(c2, H // L, k_rem)

      def cls2s(x):
        q = q_spec(x)
        return q < b2s, q == b2s, None

      m3s, sacc2s = compact_pass(buf_v, row_v, jnp.where(commit, m_int, 0),
                                 cls2s, None)
      contrib_s = lax.cond(m3s <= L, fast_tail, slow_tail, m3s, k_rem - nb2s,
                           sacc_a + sacc2s)

      m_fb = jnp.where(commit, jnp.int32(0), jnp.int32(P))
      nh_fb = jnp.where(commit, 0, HPAD // L)
      hist_pass(row_v, m_fb, lambda x: _srl(r2(x), 10), c1)
      b1, nb1 = find_count(c1, nh_fb, jnp.int32(K))
      base1 = b1 * H

      def cls1(x):
        d = r2(x) - base1
        sel = lax.bitcast_convert_type(d, jnp.uint32) < jnp.uint32(H)
        return d < 0, sel, d

      m2, sacc1 = compact_pass(row_v, buf_v, m_fb, cls1, c2)

      b2, nb2 = find_count(c2, nh_fb, K - nb1)
      base2 = base1 + b2

      def cls2(x):
        d = r2(x) - base2
        return d < 0, d == 0, None

      m3, sacc2 = compact_pass(buf_v, row_v, m2, cls2, None)

      k_rem2 = K - nb1 - nb2
      contrib_f = lax.cond(m3 <= L, fast_tail, slow_tail, m3, k_rem2,
                           sacc1 + sacc2)

      contrib = jnp.where(commit, contrib_s, contrib_f)
      return jnp.where(iota == r, contrib, contrib_acc)

    contrib_acc = lax.fori_loop(0, rows_per_w, row_body, zeros_f)
    outv[...] = contrib_acc
    pltpu.sync_copy(outv, out_hbm.at[pl.ds(wid * L, L)])

  return sc_kernel


def _tc_mean(x_ref, o_ref, *, scale):
  o_ref[...] = jnp.sum(x_ref[...], keepdims=True).reshape(1, 1) * scale


def kernel(loss):
  B = loss.shape[0]
  P = loss.reshape(B, -1).shape[1]
  K = int(0.7 * P)
  sc_kernel = _make_sc_kernel(B, P, K)
  partials = sc_kernel(loss.reshape(B, -1))
  out = pl.pallas_call(
      functools.partial(_tc_mean, scale=1.0 / (B * K)),
      out_shape=jax.ShapeDtypeStruct((1, 1), jnp.float32),
  )(partials.reshape(4, NW * L // 4))
  return out[0, 0]


# speedup vs baseline: 1.4158x; 1.0018x over previous
import functools

import jax
import jax.numpy as jnp
from jax import lax
from jax.experimental import pallas as pl
from jax.experimental.pallas import tpu as pltpu
from jax.experimental.pallas import tpu_sc as plsc

NC = 2
NS = 16
NW = NC * NS
L = 16
U = 4

H = 1024
HPAD = H + 16
HB3 = 2048
SPEC_LO = 0.68
SPEC_HI = 0.72


def _srl(x, n):
  return lax.shift_right_logical(x, jnp.full(jnp.shape(x), n, jnp.int32))


def _bits(x):
  return lax.bitcast_convert_type(x, jnp.int32)


def _make_sc_kernel(B, P, K):
  rows_per_w = B // NW
  mesh = plsc.VectorSubcoreMesh(core_axis_name="c", subcore_axis_name="s")

  @functools.partial(
      pl.kernel,
      out_type=jax.ShapeDtypeStruct((NW * L,), jnp.float32),
      mesh=mesh,
      compiler_params=pltpu.CompilerParams(needs_layout_passes=False),
      scratch_types=[
          pltpu.VMEM((P,), jnp.float32),
          pltpu.VMEM((P,), jnp.float32),
          pltpu.VMEM((P,), jnp.float32),
          pltpu.VMEM((HPAD,), jnp.int32),
          pltpu.VMEM((HPAD,), jnp.int32),
          pltpu.VMEM((HB3,), jnp.int32),
          pltpu.VMEM((L,), jnp.float32),
          pltpu.SemaphoreType.DMA,
      ],
  )
  def sc_kernel(loss_hbm, out_hbm, row_v, row_b, buf_v, c1, c2, c3, outv,
                dma_sem):
    cid = lax.axis_index("c")
    sid = lax.axis_index("s")
    wid = sid * NC + cid
    iota = lax.iota(jnp.int32, L)
    ones_i = jnp.ones((L,), jnp.int32)
    zeros_i = jnp.zeros((L,), jnp.int32)
    zeros_f = jnp.zeros((L,), jnp.float32)
    last_lane = jnp.full((L,), L - 1, jnp.int32)

    def bc_last(v):
      return v.at[last_lane].get(mode="promise_in_bounds")

    def zero_ref(ref, n):
      def zbody(i, _):
        ref[pl.ds(i * L, L)] = zeros_i
        return 0

      lax.fori_loop(0, n // L, zbody, 0)

    zero_ref(c1, HPAD)
    zero_ref(c2, HPAD)
    zero_ref(c3, HB3)

    def find_count(c_ref, nchunks, k_rem):
      def fbody(i, carry):
        cum, found, cum_sel, base_sel, c_sel = carry
        c = c_ref[pl.ds(i * L, L)]
        tot = jnp.sum(c)
        c_ref[pl.ds(i * L, L)] = zeros_i
        hit = jnp.logical_and(jnp.logical_not(found), (cum + tot) >= k_rem)
        cum_sel = jnp.where(hit, cum, cum_sel)
        base_sel = jnp.where(hit, i * L, base_sel)
        c_sel = jnp.where(hit, c, c_sel)
        return (cum + tot, jnp.logical_or(found, hit), cum_sel, base_sel,
                c_sel)

      init = (jnp.int32(0), jnp.bool_(False), jnp.int32(0), jnp.int32(0),
              zeros_i)
      _, _, cum_sel, base_sel, c_sel = lax.fori_loop(0, nchunks, fbody, init)
      scan_c = plsc.cumsum(c_sel)
      cross = (cum_sel + scan_c) >= k_rem
      lane = jnp.min(jnp.where(cross, iota, L - 1))
      nb = cum_sel + jnp.sum(jnp.where(iota < lane, c_sel, 0))
      return base_sel + lane, nb

    def hist_pass(src, m, qfn, c_ref):
      full = isinstance(m, int)
      nch = m // L if full else _srl(m + (L - 1), 4)

      @plsc.parallel_loop(0, nch, unroll=2 * U)
      def _(i):
        base = i * L
        x = src[pl.ds(base, L)]
        b = qfn(x)
        if full:
          plsc.addupdate_scatter(c_ref, [b], ones_i)
        else:
          valid = (base + iota) < m
          plsc.addupdate_scatter(c_ref, [b], ones_i, mask=valid)

    def compact_pass(src, dst, m, classify, c_next, count_below=False):
      full = isinstance(m, int)
      nit = m // (L * U) if full else _srl(m + (L * U - 1), 6)
      init = (zeros_i, zeros_f, zeros_i) if count_below else (zeros_i, zeros_f)

      @plsc.parallel_loop(0, nit, unroll=2, carry=init)
      def res(i, carry):
        offv, sacc = carry[0], carry[1]
        cb = carry[2] if count_below else None
        xs, sels, pms, cnts, keys = [], [], [], [], []
        for u in range(U):
          base = (i * U + u) * L
          x = src[pl.ds(base, L)]
          below, sel, nkey = classify(x)
          if not full:
            valid = (base + iota) < m
            below = jnp.logical_and(valid, below)
            sel = jnp.logical_and(valid, sel)
          sacc = sacc + jnp.where(below, x, jnp.float32(0))
          if count_below:
            cb = cb + below.astype(jnp.int32)
          pm = plsc.cumsum(sel.astype(jnp.int32))
          xs.append(x)
          sels.append(sel)
          pms.append(pm)
          cnts.append(bc_last(pm))
          keys.append(nkey)
        c01 = cnts[0] + cnts[1]
        offs = [offv, offv + cnts[0], offv + c01, offv + c01 + cnts[2]]
        for u in range(U):
          plsc.store_scatter(dst, [offs[u] + pms[u] - 1], xs[u], mask=sels[u])
          if c_next is not None:
            plsc.addupdate_scatter(c_next, [keys[u]], ones_i, mask=sels[u])
        if count_below:
          return (offs[3] + cnts[3], sacc, cb)
        return (offs[3] + cnts[3], sacc)

      if count_below:
        offv, sacc, cb = res
        return jnp.max(offv), sacc, jnp.sum(cb)
      offv, sacc = res
      return jnp.max(offv), sacc

    def below_sum(src, m, t_bits):
      @plsc.parallel_loop(0, _srl(m + (L - 1), 4), unroll=U, carry=zeros_f)
      def res(i, sacc):
        base = i * L
        x = src[pl.ds(base, L)]
        below = jnp.logical_and((base + iota) < m, _bits(x) < t_bits)
        return sacc + jnp.where(below, x, jnp.float32(0))

      return res

    r2 = lambda x: (x * jnp.float32(H * H)).astype(jnp.int32)
    qb1 = lambda x: _srl(_bits(x), 21) & 0x3FF
    qb2 = lambda x: _srl(_bits(x), 11) & 0x3FF
    qb3 = lambda x: _bits(x) & 0x7FF

    def classify_q(qfn, bs, nqfn):
      def f(x):
        q = qfn(x)
        return q < bs, q == bs, None if nqfn is None else nqfn(x)

      return f

    def make_fast_tail(src):
      def fast_tail(m3, k_rem2, sacc_lin):
        x = src[pl.ds(0, L)]
        xk = jnp.where(iota < m3, x, jnp.float32(2.0))
        xs = lax.sort(xk)
        tail = jnp.sum(jnp.where(iota < k_rem2, xs, jnp.float32(0)))
        return jnp.sum(sacc_lin) + tail

      return fast_tail

    def make_slow_tail(src):
      def slow_tail(m3, k_rem2, sacc_lin):
        live = m3 > L
        m3 = jnp.where(live, m3, 0)
        nh = jnp.where(live, HPAD // L, 0)
        nh3 = jnp.where(live, HB3 // L, 0)
        hist_pass(src, m3, qb1, c1)
        b1, nb1 = find_count(c1, nh, k_rem2)
        m4, sacc3 = compact_pass(src, buf_v, m3, classify_q(qb1, b1, qb2), c2)
        b2, nb2 = find_count(c2, nh, k_rem2 - nb1)
        m5, sacc4 = compact_pass(buf_v, src, m4, classify_q(qb2, b2, qb3), c3)
        b3, nb3 = find_count(c3, nh3, k_rem2 - nb1 - nb2)
        t_bits = (b1 << 21) | (b2 << 11) | b3
        sacc5 = below_sum(src, m5, t_bits)
        t_vec = lax.bitcast_convert_type(jnp.full((L,), t_bits, jnp.int32),
                                         jnp.float32)
        rem = (k_rem2 - nb1 - nb2 - nb3).astype(jnp.float32)
        contrib_v = (sacc_lin + sacc3 + sacc4 + sacc5
                     + rem * t_vec * jnp.float32(1.0 / L))
        return jnp.sum(contrib_v)

      return slow_tail

    spec_scale = jnp.float32(H / (SPEC_HI - SPEC_LO))
    spec_shift = jnp.float32(-SPEC_LO * H / (SPEC_HI - SPEC_LO))

    def q_spec(x):
      return (x * spec_scale + spec_shift).astype(jnp.int32)

    def cls_spec(x):
      q = q_spec(x)
      sel = lax.bitcast_convert_type(q, jnp.uint32) < jnp.uint32(H)
      return q < 0, sel, q

    def compute_row(src):
      m_int, sacc_a, n_below = compact_pass(src, buf_v, P, cls_spec, c2,
                                            count_below=True)
      k_rem = K - n_below
      commit = jnp.logical_and(k_rem >= 1, k_rem <= m_int)

      b2s, nb2s = find_count(c2, H // L, k_rem)

      def cls2s(x):
        q = q_spec(x)
        return q < b2s, q == b2s, None

      m3s, sacc2s = compact_pass(buf_v, src, jnp.where(commit, m_int, 0),
                                 cls2s, None)
      contrib_s = lax.cond(m3s <= L, make_fast_tail(src), make_slow_tail(src),
                           m3s, k_rem - nb2s, sacc_a + sacc2s)

      m_fb = jnp.where(commit, jnp.int32(0), jnp.int32(P))
      nh_fb = jnp.where(commit, 0, HPAD // L)
      hist_pass(src, m_fb, lambda x: _srl(r2(x), 10), c1)
      b1, nb1 = find_count(c1, nh_fb, jnp.int32(K))
      base1 = b1 * H

      def cls1(x):
        d = r2(x) - base1
        sel = lax.bitcast_convert_type(d, jnp.uint32) < jnp.uint32(H)
        return d < 0, sel, d

      m2, sacc1 = compact_pass(src, buf_v, m_fb, cls1, c2)

      b2, nb2 = find_count(c2, nh_fb, K - nb1)
      base2 = base1 + b2

      def cls2(x):
        d = r2(x) - base2
        return d < 0, d == 0, None

      m3, sacc2 = compact_pass(buf_v, src, m2, cls2, None)

      k_rem2 = K - nb1 - nb2
      contrib_f = lax.cond(m3 <= L, make_fast_tail(src), make_slow_tail(src),
                           m3, k_rem2, sacc1 + sacc2)

      return jnp.where(commit, contrib_s, contrib_f)

    row0 = wid * rows_per_w
    pltpu.sync_copy(loss_hbm.at[row0], row_v)

    def pair_body(j, contrib_acc):
      r0 = 2 * j
      cp_b = pltpu.async_copy(loss_hbm.at[row0 + r0 + 1], row_b, dma_sem)
      contrib0 = compute_row(row_v)
      cp_b.wait()
      nxt = jnp.minimum(r0 + 2, rows_per_w - 1)
      cp_a = pltpu.async_copy(loss_hbm.at[row0 + nxt], row_v, dma_sem)
      contrib1 = compute_row(row_b)
      cp_a.wait()
      acc = jnp.where(iota == r0, contrib0, contrib_acc)
      return jnp.where(iota == r0 + 1, contrib1, acc)

    contrib_acc = lax.fori_loop(0, rows_per_w // 2, pair_body, zeros_f)
    outv[...] = contrib_acc
    pltpu.sync_copy(outv, out_hbm.at[pl.ds(wid * L, L)])

  return sc_kernel


def _tc_mean(x_ref, o_ref, *, scale):
  o_ref[...] = jnp.sum(x_ref[...], keepdims=True).reshape(1, 1) * scale


def kernel(loss):
  B = loss.shape[0]
  P = loss.reshape(B, -1).shape[1]
  K = int(0.7 * P)
  sc_kernel = _make_sc_kernel(B, P, K)
  partials = sc_kernel(loss.reshape(B, -1))
  out = pl.pallas_call(
      functools.partial(_tc_mean, scale=1.0 / (B * K)),
      out_shape=jax.ShapeDtypeStruct((1, 1), jnp.float32),
  )(partials.reshape(4, NW * L // 4))
  return out[0, 0]
